# baseline (speedup 1.0000x reference)
import functools

import jax
import jax.numpy as jnp
import numpy as np
from jax import lax
from jax.experimental import pallas as pl
from jax.experimental.pallas import tpu as pltpu
from jax.experimental.pallas import tpu_sc as plsc

_VOCAB = 100000
_EMB = 50
_B = 4096
_L = 200
_NC = 2
_NS = 16
_NW = _NC * _NS
_ROWS = _B // _NW
_KR = 4
_NBLK = _ROWS // _KR
_IDX_CHUNK = 100
_NCHUNK = _L // _IDX_CHUNK
_EMBP = 64
_PREP_BLK = 10000


def _sc_body(x_hbm, tab_hbm, w1_hbm, out_hbm, idx_v, rows_v, w1_v, outb_v,
             sem0, sem1, sem_idx, sem_out):
    cid = lax.axis_index("c")
    sid = lax.axis_index("s")
    wid = sid * _NC + cid
    base = wid * _ROWS

    pltpu.sync_copy(w1_hbm, w1_v)

    sems = (sem0, sem1)

    def fetch_idx(blk, ib):
        pltpu.async_copy(x_hbm.at[wid * _NBLK + blk], idx_v.at[ib],
                         sem_idx)

    def wait_idx(blk, ib):
        pltpu.make_async_copy(x_hbm.at[wid * _NBLK + blk], idx_v.at[ib],
                              sem_idx).wait()

    def fire(b):
        for r in range(_KR):
            for c in range(_NCHUNK):
                pltpu.async_copy(
                    tab_hbm.at[idx_v.at[b, r, c]],
                    rows_v.at[b, r, pl.ds(c * _IDX_CHUNK, _IDX_CHUNK)],
                    sems[b])

    def wait(b):
        for r in range(_KR):
            for c in range(_NCHUNK):
                pltpu.make_async_copy(
                    tab_hbm.at[idx_v.at[b, r, c]],
                    rows_v.at[b, r, pl.ds(c * _IDX_CHUNK, _IDX_CHUNK)],
                    sems[b]).wait()

    def wait_out(blk, b):
        pltpu.make_async_copy(
            outb_v.at[b], out_hbm.at[pl.ds(base + blk * _KR, _KR)],
            sem_out).wait()

    fetch_idx(0, 0)
    wait_idx(0, 0)
    fire(0)
    fetch_idx(1, 1)

    def pair_body(i, carry):
        for b in range(2):
            blk = 2 * i + b
            wait(b)
            nblk = blk + 1
            nnblk = blk + 2

            @pl.when(nblk < _NBLK)
            def _():
                wait_idx(nblk, 1 - b)
                fire(1 - b)

            @pl.when(nnblk < _NBLK)
            def _():
                fetch_idx(nnblk, b)

            def tok_body(t, accs):
                accs = list(accs)
                woff = t * _EMBP
                for h in range(2):
                    offa = woff + 32 * h
                    wa = [w1_v[f, pl.ds(offa, 16)] for f in range(4)]
                    wb = [w1_v[f, pl.ds(offa + 16, 16)] for f in range(4)]
                    for r in range(_KR):
                        dv = rows_v[b, r, t, pl.ds(32 * h, 32)]
                        da, db = plsc.unpack(
                            dv, format=plsc.PackFormat.INTERLEAVED)
                        for f in range(4):
                            accs[4 * r + f] = (accs[4 * r + f]
                                               + da * wa[f] + db * wb[f])
                return tuple(accs)

            z = jnp.zeros((16,), jnp.float32)
            accs = lax.fori_loop(0, _L, tok_body, (z,) * (4 * _KR))

            @pl.when(blk >= 2)
            def _():
                wait_out(blk - 2, b)

            for r in range(_KR):
                for f in range(4):
                    outb_v[b, r, pl.ds(16 * f, 16)] = accs[4 * r + f]
            pltpu.async_copy(
                outb_v.at[b], out_hbm.at[pl.ds(base + blk * _KR, _KR)],
                sem_out)
        return carry

    lax.fori_loop(0, _NBLK // 2, pair_body, 0)
    wait_out(_NBLK - 2, 0)
    wait_out(_NBLK - 1, 1)


_sc_first_layer = functools.partial(
    pl.kernel,
    out_type=jax.ShapeDtypeStruct((_B, 4 * 16), jnp.float32),
    mesh=plsc.VectorSubcoreMesh(
        core_axis_name="c", subcore_axis_name="s",
        num_cores=_NC, num_subcores=_NS),
    scratch_types=[
        pltpu.VMEM((2, _KR, _NCHUNK, _IDX_CHUNK), jnp.int32),
        pltpu.VMEM((2, _KR, _L, _EMBP), jnp.bfloat16),
        pltpu.VMEM((4, _L * _EMBP), jnp.float32),
        pltpu.VMEM((2, _KR, 4 * 16), jnp.float32),
        pltpu.SemaphoreType.DMA,
        pltpu.SemaphoreType.DMA,
        pltpu.SemaphoreType.DMA,
        pltpu.SemaphoreType.DMA,
    ],
    compiler_params=pltpu.CompilerParams(use_tc_tiling_on_sc=False,
                                         needs_layout_passes=False),
)(_sc_body)


def _head_body(p_ref, msum_ref, b1_ref, w2_ref, b2_ref, w3_ref, b3_ref,
               w4_ref, b4_ref, o_ref):
    h = lax.dot(p_ref[:], msum_ref[:],
                preferred_element_type=jnp.float32) + b1_ref[:]
    h = jnp.maximum(h, 0.0)
    h = jnp.maximum(
        lax.dot(h, w2_ref[:], preferred_element_type=jnp.float32)
        + b2_ref[:], 0.0)
    h = jnp.maximum(
        lax.dot(h, w3_ref[:], preferred_element_type=jnp.float32)
        + b3_ref[:], 0.0)
    logits = lax.dot(h, w4_ref[:], preferred_element_type=jnp.float32) \
        + b4_ref[:]
    m = jnp.max(logits, axis=1, keepdims=True)
    lse = m + jnp.log(jnp.sum(jnp.exp(logits - m), axis=1, keepdims=True))
    o_ref[:] = logits - lse


def kernel(X, emb_table, W1, b1, W2, b2, W3, b3, W4, b4):
    x_r = X.astype(jnp.int32).reshape(_B // _KR, _KR, _NCHUNK, _IDX_CHUNK)

    tab_bf = jnp.pad(emb_table.astype(jnp.bfloat16),
                     ((0, 0), (0, _EMBP - _EMB)))

    q = np.arange(_EMBP)
    elem = 32 * (q // 32) + 2 * (q % 16) + (q % 32) // 16
    w1_r = W1.reshape(_L, _EMB, 4)
    w1_p = jnp.concatenate(
        [w1_r, jnp.zeros((_L, _EMBP - _EMB, 4), jnp.float32)], axis=1)
    w1_t = w1_p[:, elem, :].transpose(2, 0, 1).reshape(4, _L * _EMBP)

    partial = _sc_first_layer(x_r, tab_bf, w1_t)

    msum = jnp.repeat(jnp.eye(4, dtype=jnp.float32), 16, axis=0)
    out = pl.pallas_call(
        _head_body,
        out_shape=jax.ShapeDtypeStruct((_B, 2), jnp.float32),
    )(partial, msum, b1.reshape(1, 4), W2, b2.reshape(1, 3),
      W3, b3.reshape(1, 3), W4, b4.reshape(1, 2))
    return out


# speedup vs baseline: 1.0008x; 1.0008x over previous
import functools

import jax
import jax.numpy as jnp
import numpy as np
from jax import lax
from jax.experimental import pallas as pl
from jax.experimental.pallas import tpu as pltpu
from jax.experimental.pallas import tpu_sc as plsc

_VOCAB = 100000
_EMB = 50
_B = 4096
_L = 200
_NC = 2
_NS = 16
_NW = _NC * _NS
_ROWS = _B // _NW
_KR = 4
_NBLK = _ROWS // _KR
_IDX_CHUNK = 100
_NCHUNK = _L // _IDX_CHUNK
_EMBP = 64


def _sc_body(x_hbm, tab_hbm, w1_hbm, out_hbm, idx_v, rows_v, w1_v, outb_v,
             sem0, sem1, sem_idx, sem_out):
    cid = lax.axis_index("c")
    sid = lax.axis_index("s")
    wid = sid * _NC + cid
    base = wid * _ROWS

    pltpu.sync_copy(w1_hbm, w1_v)

    sems = (sem0, sem1)

    def fetch_idx(blk, ib):
        pltpu.async_copy(x_hbm.at[wid * _NBLK + blk], idx_v.at[ib],
                         sem_idx)

    def wait_idx(blk, ib):
        pltpu.make_async_copy(x_hbm.at[wid * _NBLK + blk], idx_v.at[ib],
                              sem_idx).wait()

    def fire(b):
        for r in range(_KR):
            for c in range(_NCHUNK):
                pltpu.async_copy(
                    tab_hbm.at[idx_v.at[b, r, c]],
                    rows_v.at[b, r, pl.ds(c * _IDX_CHUNK, _IDX_CHUNK)],
                    sems[b])

    def wait(b):
        for r in range(_KR):
            for c in range(_NCHUNK):
                pltpu.make_async_copy(
                    tab_hbm.at[idx_v.at[b, r, c]],
                    rows_v.at[b, r, pl.ds(c * _IDX_CHUNK, _IDX_CHUNK)],
                    sems[b]).wait()

    def wait_out(blk, b):
        pltpu.make_async_copy(
            outb_v.at[b], out_hbm.at[pl.ds(base + blk * _KR, _KR)],
            sem_out).wait()

    fetch_idx(0, 0)
    wait_idx(0, 0)
    fire(0)
    fetch_idx(1, 1)

    def pair_body(i, carry):
        for b in range(2):
            blk = 2 * i + b
            wait(b)
            nblk = blk + 1
            nnblk = blk + 2

            @pl.when(nblk < _NBLK)
            def _():
                wait_idx(nblk, 1 - b)
                fire(1 - b)

            @pl.when(nnblk < _NBLK)
            def _():
                fetch_idx(nnblk, b)

            def tok_body(t, accs):
                accs = list(accs)
                woff = t * _EMBP
                for h in range(2):
                    offa = woff + 32 * h
                    wa = [w1_v[f, pl.ds(offa, 16)] for f in range(4)]
                    wb = [w1_v[f, pl.ds(offa + 16, 16)] for f in range(4)]
                    for r in range(_KR):
                        dv = rows_v[b, r, t, pl.ds(32 * h, 32)]
                        da, db = plsc.unpack(
                            dv, format=plsc.PackFormat.INTERLEAVED)
                        for f in range(4):
                            accs[4 * r + f] = (accs[4 * r + f]
                                               + da * wa[f] + db * wb[f])
                return tuple(accs)

            z = jnp.zeros((16,), jnp.float32)
            accs = lax.fori_loop(0, _L, tok_body, (z,) * (4 * _KR))

            @pl.when(blk >= 2)
            def _():
                wait_out(blk - 2, b)

            for r in range(_KR):
                for f in range(4):
                    outb_v[b, r, pl.ds(16 * f, 16)] = accs[4 * r + f]
            pltpu.async_copy(
                outb_v.at[b], out_hbm.at[pl.ds(base + blk * _KR, _KR)],
                sem_out)
        return carry

    lax.fori_loop(0, _NBLK // 2, pair_body, 0)
    wait_out(_NBLK - 2, 0)
    wait_out(_NBLK - 1, 1)


_sc_first_layer = functools.partial(
    pl.kernel,
    out_type=jax.ShapeDtypeStruct((_B, 4 * 16), jnp.float32),
    mesh=plsc.VectorSubcoreMesh(
        core_axis_name="c", subcore_axis_name="s",
        num_cores=_NC, num_subcores=_NS),
    scratch_types=[
        pltpu.VMEM((2, _KR, _NCHUNK, _IDX_CHUNK), jnp.int32),
        pltpu.VMEM((2, _KR, _L, _EMBP), jnp.bfloat16),
        pltpu.VMEM((4, _L * _EMBP), jnp.float32),
        pltpu.VMEM((2, _KR, 4 * 16), jnp.float32),
        pltpu.SemaphoreType.DMA,
        pltpu.SemaphoreType.DMA,
        pltpu.SemaphoreType.DMA,
        pltpu.SemaphoreType.DMA,
    ],
    compiler_params=pltpu.CompilerParams(use_tc_tiling_on_sc=False,
                                         needs_layout_passes=False),
)(_sc_body)


def _head_body(p_ref, msum_ref, b1_ref, w2_ref, b2_ref, w3_ref, b3_ref,
               w4_ref, b4_ref, o_ref):
    h = lax.dot(p_ref[:], msum_ref[:],
                preferred_element_type=jnp.float32) + b1_ref[:]
    h = jnp.maximum(h, 0.0)
    h = jnp.maximum(
        lax.dot(h, w2_ref[:], preferred_element_type=jnp.float32)
        + b2_ref[:], 0.0)
    h = jnp.maximum(
        lax.dot(h, w3_ref[:], preferred_element_type=jnp.float32)
        + b3_ref[:], 0.0)
    logits = lax.dot(h, w4_ref[:], preferred_element_type=jnp.float32) \
        + b4_ref[:]
    m = jnp.max(logits, axis=1, keepdims=True)
    lse = m + jnp.log(jnp.sum(jnp.exp(logits - m), axis=1, keepdims=True))
    o_ref[:] = logits - lse


def kernel(X, emb_table, W1, b1, W2, b2, W3, b3, W4, b4):
    x_r = X.astype(jnp.int32).reshape(_B // _KR, _KR, _NCHUNK, _IDX_CHUNK)

    tab_bf = jnp.pad(emb_table.astype(jnp.bfloat16),
                     ((0, 0), (0, _EMBP - _EMB)))

    q = np.arange(_EMBP)
    elem = 32 * (q // 32) + 2 * (q % 16) + (q % 32) // 16
    w1_r = W1.reshape(_L, _EMB, 4)
    w1_p = jnp.concatenate(
        [w1_r, jnp.zeros((_L, _EMBP - _EMB, 4), jnp.float32)], axis=1)
    w1_t = w1_p[:, elem, :].transpose(2, 0, 1).reshape(4, _L * _EMBP)

    partial = _sc_first_layer(x_r, tab_bf, w1_t)

    msum = jnp.repeat(jnp.eye(4, dtype=jnp.float32), 16, axis=0)
    out = pl.pallas_call(
        _head_body,
        out_shape=jax.ShapeDtypeStruct((_B, 2), jnp.float32),
    )(partial, msum, b1.reshape(1, 4), W2, b2.reshape(1, 3),
      W3, b3.reshape(1, 3), W4, b4.reshape(1, 2))
    return out
